# Initial kernel scaffold; baseline (speedup 1.0000x reference)
import jax
import jax.numpy as jnp
from jax.experimental import pallas as pl


def kernel(t, xs, h_init, emb_W, emb_b, out_W, out_b, eW1, eb1, eW2, eb2, nW1, nb1, nW2, nb2, cW1, cb1, cW2, rows, cols):
    raise NotImplementedError("write your pallas kernel here")


# speedup vs baseline: 9.1039x; 9.1039x over previous
import jax
import jax.numpy as jnp
from jax.experimental import pallas as pl

_B, _P, _D, _H, _L = 1024, 22, 3, 64, 4
_PP = 24
_BB = 16
_NN = _BB * _PP
_R = _NN * _PP


def _egnn_block(t_ref, xn_ref, h0p_ref, temb_ref,
                Wa_ref, Wb_ref, wr_ref, we_ref, eb1_ref,
                eW2_ref, eb2_ref,
                cW1_ref, cb1_ref, cW2_ref,
                nW1h_ref, nW1a_ref, nb1_ref, nW2_ref, nb2_ref,
                out_ref):
    f32 = jnp.float32

    def ibcast(v):
        c = v.shape[-1]
        return jnp.broadcast_to(v.reshape(_NN, 1, c), (_NN, _PP, c)).reshape(_R, c)

    def jbcast3(v):
        c = v.shape[-1]
        return jnp.broadcast_to(v.reshape(_BB, 1, _PP, c), (_BB, _PP, _PP, c)).reshape(_NN, _PP, c)

    ii = jax.lax.broadcasted_iota(jnp.int32, (_NN, _PP, 1), 0) % _PP
    jj = jax.lax.broadcasted_iota(jnp.int32, (_NN, _PP, 1), 1)
    pairmask = ((jj < _P) & (jj != ii)).astype(f32)

    xn = xn_ref[...]
    x0 = [xn[:, d:d + 1] for d in range(_D)]

    tnode = jnp.broadcast_to(t_ref[...].reshape(_BB, 1, 1), (_BB, _PP, 1)).reshape(_NN, 1)
    h = (jnp.broadcast_to(h0p_ref[...].reshape(1, _PP, _H), (_BB, _PP, _H)).reshape(_NN, _H)
         + tnode * temb_ref[...])

    x0j3 = [jbcast3(x0[d]) for d in range(_D)]
    r0 = sum((ibcast(x0[d]) - x0j3[d].reshape(_R, 1)) ** 2 for d in range(_D))

    x = list(x0)
    xj3 = x0j3
    for l in range(_L):
        xi = [ibcast(x[d]) for d in range(_D)]
        diff = [xi[d] - xj3[d].reshape(_R, 1) for d in range(_D)]
        radial = diff[0] ** 2 + diff[1] ** 2 + diff[2] ** 2
        norm = jnp.sqrt(radial + 1e-8)

        A = jnp.dot(h, Wa_ref[l], preferred_element_type=f32)
        Bv = jnp.dot(h, Wb_ref[l], preferred_element_type=f32)
        z = (ibcast(A) + jbcast3(Bv).reshape(_R, _H)
             + radial * wr_ref[l] + r0 * we_ref[l] + eb1_ref[l])
        ef1 = jax.nn.silu(z)
        ef2 = jax.nn.silu(jnp.dot(ef1, eW2_ref[l], preferred_element_type=f32) + eb2_ref[l])
        c1 = jax.nn.silu(jnp.dot(ef2, cW1_ref[l], preferred_element_type=f32) + cb1_ref[l])
        cm = jnp.dot(c1, cW2_ref[l], preferred_element_type=f32)

        w3 = (cm / (norm + 1.0)).reshape(_NN, _PP, 1) * pairmask
        wsum = jnp.sum(w3, axis=1)
        for d in range(_D):
            td = jnp.sum(w3 * xj3[d], axis=1)
            x[d] = x[d] + x[d] * wsum - td
        xj3 = [jbcast3(x[d]) for d in range(_D)]

        if l < _L - 1:
            agg = jnp.sum(ef2.reshape(_NN, _PP, _H) * pairmask, axis=1)
            m1 = jax.nn.silu(jnp.dot(h, nW1h_ref[l], preferred_element_type=f32)
                             + jnp.dot(agg, nW1a_ref[l], preferred_element_type=f32)
                             + nb1_ref[l])
            h = h + jnp.dot(m1, nW2_ref[l], preferred_element_type=f32) + nb2_ref[l]

    nodemask = (jax.lax.broadcasted_iota(jnp.int32, (_NN, 1), 0) % _PP < _P).astype(f32)
    cols = []
    for d in range(_D):
        vd = (x[d] - x0[d]) * nodemask
        mean = jnp.sum(vd.reshape(_BB, _PP, 1), axis=1) * (1.0 / _P)
        mean_n = jnp.broadcast_to(mean.reshape(_BB, 1, 1), (_BB, _PP, 1)).reshape(_NN, 1)
        cols.append((vd - mean_n) * nodemask)
    out_ref[...] = jnp.concatenate(cols, axis=1)


def kernel(t, xs, h_init, emb_W, emb_b, out_W, out_b, eW1, eb1, eW2, eb2,
           nW1, nb1, nW2, nb2, cW1, cb1, cW2, rows, cols):
    f32 = jnp.float32
    xpad = jnp.pad(xs.reshape(_B, _P, _D), ((0, 0), (0, _PP - _P), (0, 0)))
    xn = xpad.reshape(_B * _PP, _D)
    h0p = jnp.pad(h_init @ emb_W[:2] + emb_b, ((0, _PP - _P), (0, 0)))
    temb = emb_W[4:5]
    Wa = eW1[:, :_H]
    Wb = eW1[:, _H:2 * _H]
    wr = eW1[:, 2 * _H:2 * _H + 1]
    we = eW1[:, 2 * _H + 1:]
    nW1h = nW1[:, :_H]
    nW1a = nW1[:, _H:]
    eb1r = eb1.reshape(_L, 1, _H)
    eb2r = eb2.reshape(_L, 1, _H)
    cb1r = cb1.reshape(_L, 1, _H)
    nb1r = nb1.reshape(_L, 1, _H)
    nb2r = nb2.reshape(_L, 1, _H)

    def full(a):
        return pl.BlockSpec(a.shape, lambda i: (0,) * a.ndim)

    out = pl.pallas_call(
        _egnn_block,
        grid=(_B // _BB,),
        in_specs=[
            pl.BlockSpec((_BB, 1), lambda i: (i, 0)),
            pl.BlockSpec((_NN, _D), lambda i: (i, 0)),
            full(h0p), full(temb),
            full(Wa), full(Wb), full(wr), full(we), full(eb1r),
            full(eW2), full(eb2r),
            full(cW1), full(cb1r), full(cW2),
            full(nW1h), full(nW1a), full(nb1r), full(nW2), full(nb2r),
        ],
        out_specs=pl.BlockSpec((_NN, _D), lambda i: (i, 0)),
        out_shape=jax.ShapeDtypeStruct((_B * _PP, _D), f32),
    )(t, xn, h0p, temb, Wa, Wb, wr, we, eb1r, eW2, eb2r,
      cW1, cb1r, cW2, nW1h, nW1a, nb1r, nW2, nb2r)

    return out.reshape(_B, _PP, _D)[:, :_P, :].reshape(_B, _P * _D)
